# baseline (speedup 1.0000x reference)
import dataclasses
import functools

import jax
import jax.numpy as jnp
from jax import lax
from jax.experimental import pallas as pl
from jax.experimental.pallas import tpu as pltpu
from jax.experimental.pallas import tpu_sc as plsc

TT = 32768
DD = 128
NSEG = 16
LANES = 16
NVEC = DD // LANES
NC = 2
NS = 16
NW = NC * NS
RPW = TT // NW
C1 = 256
C3 = 128
CU_PAD = 32


def _sc_compiler_params():
    cp = pltpu.CompilerParams()
    if "needs_layout_passes" in pltpu.CompilerParams.__dataclass_fields__:
        cp = dataclasses.replace(cp, needs_layout_passes=False)
    return cp


def _cu_scalars(cu_v):
    cu_vec = cu_v[pl.ds(0, LANES)]
    lane_ids = lax.iota(jnp.int32, LANES)
    cus = [jnp.int32(0)]
    for s in range(1, NSEG):
        cus.append(jnp.sum(jnp.where(lane_ids == s, cu_vec, 0)))
    cus.append(jnp.int32(TT))
    return cus


def _pass1_body(flat_hbm, cu_hbm, part_hbm, buf, acc, cu_v, isem0, isem1):
    isems = (isem0, isem1)
    wid = lax.axis_index("c") * NS + lax.axis_index("s")
    base = wid * RPW
    pltpu.sync_copy(cu_hbm, cu_v)
    cus = _cu_scalars(cu_v)
    zero = jnp.zeros((LANES,), jnp.float32)
    for s in range(NSEG):
        for j in range(NVEC):
            acc[s, pl.ds(j * LANES, LANES)] = zero
    nch = RPW // C1
    pltpu.async_copy(flat_hbm.at[pl.ds(base, C1)], buf.at[0], isems[0])
    for k in range(nch):
        b = k % 2
        chunk_lo = base + k * C1
        if k + 1 < nch:
            nb = (k + 1) % 2
            pltpu.async_copy(flat_hbm.at[pl.ds(chunk_lo + C1, C1)],
                             buf.at[nb], isems[nb])
        pltpu.make_async_copy(flat_hbm.at[pl.ds(chunk_lo, C1)],
                              buf.at[b], isems[b]).wait()
        for s in range(NSEG):
            lo = jnp.clip(cus[s], chunk_lo, chunk_lo + C1) - chunk_lo
            hi = jnp.clip(cus[s + 1], chunk_lo, chunk_lo + C1) - chunk_lo

            @pl.when(hi > lo)
            def _run(b=b, s=s, lo=lo, hi=hi):
                def row(i, c):
                    return tuple(c[j] + buf[b, i, pl.ds(j * LANES, LANES)]
                                 for j in range(NVEC))

                sums = plsc.parallel_loop(
                    lo, hi, unroll=2,
                    carry=tuple(zero for _ in range(NVEC)))(row)
                for j in range(NVEC):
                    sl = pl.ds(j * LANES, LANES)
                    acc[s, sl] = acc[s, sl] + sums[j]
    for s in range(NSEG):
        pltpu.sync_copy(acc.at[s], part_hbm.at[s, wid])


def _cu_at_dyn(cu_v, idx):
    lane_ids = lax.iota(jnp.int32, LANES)
    v0 = cu_v[pl.ds(0, LANES)]
    v1 = cu_v[pl.ds(LANES, LANES)]
    return (jnp.sum(jnp.where(lane_ids == idx, v0, 0))
            + jnp.sum(jnp.where(lane_ids == idx - LANES, v1, 0)))


def _pass3_body(flat_hbm, cu_hbm, part_hbm, w_hbm, out_hbm,
                ibuf, obuf, gate_v, cu_v, pbuf, wbuf, pooled_v, grow_v,
                gate_sh, isem0, isem1, osem0, osem1):
    isems = (isem0, isem1)
    osems = (osem0, osem1)
    cid = lax.axis_index("c")
    tid = lax.axis_index("s")
    wid = cid * NS + tid
    base = wid * RPW
    nch = RPW // C3
    pltpu.async_copy(flat_hbm.at[pl.ds(base, C3)], ibuf.at[0], isems[0])
    pltpu.async_copy(flat_hbm.at[pl.ds(base + C3, C3)], ibuf.at[1], isems[1])
    pltpu.sync_copy(cu_hbm, cu_v)
    cus = _cu_scalars(cu_v)

    pltpu.sync_copy(part_hbm.at[tid], pbuf)
    pltpu.sync_copy(w_hbm, wbuf)
    zero = jnp.zeros((LANES,), jnp.float32)

    def _wsum(w, c):
        return tuple(c[j] + pbuf[w, pl.ds(j * LANES, LANES)]
                     for j in range(NVEC))

    psum = plsc.parallel_loop(0, NW, unroll=2,
                              carry=tuple(zero for _ in range(NVEC)))(_wsum)
    cnt = _cu_at_dyn(cu_v, tid + 1) - _cu_at_dyn(cu_v, tid)
    cnt_vec = jnp.zeros((LANES,), jnp.float32) + cnt.astype(jnp.float32)
    inv = 1.0 / jnp.maximum(cnt_vec, 1.0)
    for j in range(NVEC):
        pooled_v[pl.ds(j * LANES, LANES)] = psum[j] * inv

    def _fma(k, c):
        pk = plsc.load_gather(pooled_v, [jnp.full((LANES,), 0, jnp.int32) + k])
        return tuple(c[j] + pk * wbuf[k, pl.ds(j * LANES, LANES)]
                     for j in range(NVEC))

    macc = plsc.parallel_loop(0, DD, unroll=2,
                              carry=tuple(zero for _ in range(NVEC)))(_fma)
    for j in range(NVEC):
        g = 1.0 / (1.0 + jnp.exp(-macc[j]))
        grow_v[pl.ds(j * LANES, LANES)] = g
    pltpu.sync_copy(grow_v, gate_sh.at[tid])
    plsc.subcore_barrier()
    pltpu.sync_copy(gate_sh, gate_v)

    @pl.loop(0, nch, step=2)
    def _chunks(k):
        for b in range(2):
            kb = k + b
            chunk_lo = base + kb * C3
            pltpu.make_async_copy(flat_hbm.at[pl.ds(chunk_lo, C3)],
                                  ibuf.at[b], isems[b]).wait()

            @pl.when(kb >= 2)
            def _wait_out(b=b, chunk_lo=chunk_lo):
                pltpu.make_async_copy(
                    obuf.at[b], out_hbm.at[pl.ds(chunk_lo - 2 * C3, C3)],
                    osems[b]).wait()

            for s in range(NSEG):
                lo = jnp.clip(cus[s], chunk_lo, chunk_lo + C3) - chunk_lo
                hi = jnp.clip(cus[s + 1], chunk_lo, chunk_lo + C3) - chunk_lo

                @pl.when(hi > lo)
                def _run(b=b, s=s, lo=lo, hi=hi):
                    gvecs = [gate_v[s, pl.ds(j * LANES, LANES)]
                             for j in range(NVEC)]

                    def row(i):
                        for j in range(NVEC):
                            sl = pl.ds(j * LANES, LANES)
                            obuf[b, i, sl] = ibuf[b, i, sl] * gvecs[j]

                    plsc.parallel_loop(lo, hi, unroll=2)(row)

            pltpu.async_copy(obuf.at[b], out_hbm.at[pl.ds(chunk_lo, C3)],
                             osems[b])

            @pl.when(kb + 2 < nch)
            def _next_in(b=b, chunk_lo=chunk_lo):
                pltpu.async_copy(flat_hbm.at[pl.ds(chunk_lo + 2 * C3, C3)],
                                 ibuf.at[b], isems[b])

    for b in range(2):
        pltpu.make_async_copy(
            obuf.at[b], out_hbm.at[pl.ds(base + (nch - 2 + b) * C3, C3)],
            osems[b]).wait()


def kernel(flat, cu_seqlens, W):
    cu_pad = jnp.concatenate([
        cu_seqlens.astype(jnp.int32),
        jnp.full((CU_PAD - NSEG - 1,), TT, dtype=jnp.int32),
    ])

    mesh1 = plsc.VectorSubcoreMesh(core_axis_name="c", subcore_axis_name="s")
    pass1 = functools.partial(
        pl.kernel,
        out_type=jax.ShapeDtypeStruct((NSEG, NW, DD), jnp.float32),
        mesh=mesh1,
        compiler_params=_sc_compiler_params(),
        scratch_types=[
            pltpu.VMEM((2, C1, DD), jnp.float32),
            pltpu.VMEM((NSEG, DD), jnp.float32),
            pltpu.VMEM((CU_PAD,), jnp.int32),
            pltpu.SemaphoreType.DMA,
            pltpu.SemaphoreType.DMA,
        ],
    )(_pass1_body)
    partials = pass1(flat, cu_pad)

    mesh3 = plsc.VectorSubcoreMesh(core_axis_name="c", subcore_axis_name="s")
    pass3 = functools.partial(
        pl.kernel,
        out_type=jax.ShapeDtypeStruct((TT, DD), jnp.float32),
        mesh=mesh3,
        compiler_params=_sc_compiler_params(),
        scratch_types=[
            pltpu.VMEM((2, C3, DD), jnp.float32),
            pltpu.VMEM((2, C3, DD), jnp.float32),
            pltpu.VMEM((NSEG, DD), jnp.float32),
            pltpu.VMEM((CU_PAD,), jnp.int32),
            pltpu.VMEM((NW, DD), jnp.float32),
            pltpu.VMEM((DD, DD), jnp.float32),
            pltpu.VMEM((DD,), jnp.float32),
            pltpu.VMEM((DD,), jnp.float32),
            pltpu.VMEM_SHARED((NSEG, DD), jnp.float32),
            pltpu.SemaphoreType.DMA,
            pltpu.SemaphoreType.DMA,
            pltpu.SemaphoreType.DMA,
            pltpu.SemaphoreType.DMA,
        ],
    )(_pass3_body)
    return pass3(flat, cu_pad, partials, W)


# speedup vs baseline: 1.1240x; 1.1240x over previous
import dataclasses
import functools

import jax
import jax.numpy as jnp
from jax import lax
from jax.experimental import pallas as pl
from jax.experimental.pallas import tpu as pltpu
from jax.experimental.pallas import tpu_sc as plsc

TT = 32768
DD = 128
NSEG = 16
LANES = 16
NVEC = DD // LANES
NC = 2
NS = 16
NW = NC * NS
RPW = TT // NW
C1 = 256
C3 = 128
CU_PAD = 32


def _sc_compiler_params():
    cp = pltpu.CompilerParams()
    if "needs_layout_passes" in pltpu.CompilerParams.__dataclass_fields__:
        cp = dataclasses.replace(cp, needs_layout_passes=False)
    return cp


def _cu_scalars(cu_v):
    cu_vec = cu_v[pl.ds(0, LANES)]
    lane_ids = lax.iota(jnp.int32, LANES)
    cus = [jnp.int32(0)]
    for s in range(1, NSEG):
        cus.append(jnp.sum(jnp.where(lane_ids == s, cu_vec, 0)))
    cus.append(jnp.int32(TT))
    return cus


def _pass1_body(flat_hbm, cu_hbm, part_hbm, buf, zbuf, cu_v, idx_v,
                acc_sh, isem0, isem1):
    isems = (isem0, isem1)
    cid = lax.axis_index("c")
    tid = lax.axis_index("s")
    base = (cid * NS + tid) * RPW
    nch = RPW // C1
    pltpu.async_copy(flat_hbm.at[pl.ds(base, C1)], buf.at[0], isems[0])
    pltpu.async_copy(flat_hbm.at[pl.ds(base + C1, C1)], buf.at[1], isems[1])
    pltpu.sync_copy(cu_hbm, cu_v)
    cus = _cu_scalars(cu_v)
    zero = jnp.zeros((LANES,), jnp.float32)
    for j in range(NVEC):
        for s in range(NSEG):
            zbuf[s, pl.ds(j * LANES, LANES)] = zero

    @pl.when(tid == 0)
    def _zero_acc():
        pltpu.sync_copy(zbuf, acc_sh)

    plsc.subcore_barrier()
    lane_ids = lax.iota(jnp.int32, LANES)
    ngrp = C1 // LANES

    for k in range(nch):
        b = k % 2
        chunk_lo = base + k * C1
        pltpu.make_async_copy(flat_hbm.at[pl.ds(chunk_lo, C1)],
                              buf.at[b], isems[b]).wait()

        @pl.loop(0, ngrp)
        def _grp(g, chunk_lo=chunk_lo):
            row_ids = lane_ids + (chunk_lo + g * LANES)
            seg = jnp.zeros((LANES,), jnp.int32)
            for s in range(1, NSEG):
                seg = seg + jnp.where(row_ids >= cus[s], 1, 0)
            idx_v[g // 8, pl.ds((g % 8) * LANES, LANES)] = seg

        for h in range(C1 // 128):
            pltpu.sync_copy(buf.at[b, pl.ds(h * 128, 128)],
                            acc_sh.at[idx_v.at[h]], add=True)
        if k + 2 < nch:
            pltpu.async_copy(flat_hbm.at[pl.ds(chunk_lo + 2 * C1, C1)],
                             buf.at[b], isems[b])
    plsc.subcore_barrier()

    @pl.when(tid == 0)
    def _emit():
        pltpu.sync_copy(acc_sh, part_hbm.at[cid])


def _cu_at_dyn(cu_v, idx):
    lane_ids = lax.iota(jnp.int32, LANES)
    v0 = cu_v[pl.ds(0, LANES)]
    v1 = cu_v[pl.ds(LANES, LANES)]
    return (jnp.sum(jnp.where(lane_ids == idx, v0, 0))
            + jnp.sum(jnp.where(lane_ids == idx - LANES, v1, 0)))


def _pass3_body(flat_hbm, cu_hbm, part_hbm, w_hbm, out_hbm,
                ibuf, obuf, gate_v, cu_v, pbuf, wbuf, pooled_v, grow_v,
                gate_sh, isem0, isem1, osem0, osem1):
    isems = (isem0, isem1)
    osems = (osem0, osem1)
    cid = lax.axis_index("c")
    tid = lax.axis_index("s")
    wid = cid * NS + tid
    base = wid * RPW
    nch = RPW // C3
    pltpu.async_copy(flat_hbm.at[pl.ds(base, C3)], ibuf.at[0], isems[0])
    pltpu.async_copy(flat_hbm.at[pl.ds(base + C3, C3)], ibuf.at[1], isems[1])
    pltpu.sync_copy(cu_hbm, cu_v)
    cus = _cu_scalars(cu_v)

    pltpu.sync_copy(part_hbm, pbuf)
    pltpu.sync_copy(w_hbm, wbuf)
    zero = jnp.zeros((LANES,), jnp.float32)
    psum = tuple(pbuf[0, tid, pl.ds(j * LANES, LANES)]
                 + pbuf[1, tid, pl.ds(j * LANES, LANES)]
                 for j in range(NVEC))
    cnt = _cu_at_dyn(cu_v, tid + 1) - _cu_at_dyn(cu_v, tid)
    cnt_vec = jnp.zeros((LANES,), jnp.float32) + cnt.astype(jnp.float32)
    inv = 1.0 / jnp.maximum(cnt_vec, 1.0)
    for j in range(NVEC):
        pooled_v[pl.ds(j * LANES, LANES)] = psum[j] * inv

    def _fma(k, c):
        pk = plsc.load_gather(pooled_v, [jnp.full((LANES,), 0, jnp.int32) + k])
        return tuple(c[j] + pk * wbuf[k, pl.ds(j * LANES, LANES)]
                     for j in range(NVEC))

    macc = plsc.parallel_loop(0, DD, unroll=2,
                              carry=tuple(zero for _ in range(NVEC)))(_fma)
    for j in range(NVEC):
        g = 1.0 / (1.0 + jnp.exp(-macc[j]))
        grow_v[pl.ds(j * LANES, LANES)] = g
    pltpu.sync_copy(grow_v, gate_sh.at[tid])
    plsc.subcore_barrier()
    pltpu.sync_copy(gate_sh, gate_v)

    @pl.loop(0, nch, step=2)
    def _chunks(k):
        for b in range(2):
            kb = k + b
            chunk_lo = base + kb * C3
            pltpu.make_async_copy(flat_hbm.at[pl.ds(chunk_lo, C3)],
                                  ibuf.at[b], isems[b]).wait()

            @pl.when(kb >= 2)
            def _wait_out(b=b, chunk_lo=chunk_lo):
                pltpu.make_async_copy(
                    obuf.at[b], out_hbm.at[pl.ds(chunk_lo - 2 * C3, C3)],
                    osems[b]).wait()

            for s in range(NSEG):
                lo = jnp.clip(cus[s], chunk_lo, chunk_lo + C3) - chunk_lo
                hi = jnp.clip(cus[s + 1], chunk_lo, chunk_lo + C3) - chunk_lo

                @pl.when(hi > lo)
                def _run(b=b, s=s, lo=lo, hi=hi):
                    gvecs = [gate_v[s, pl.ds(j * LANES, LANES)]
                             for j in range(NVEC)]

                    def row(i):
                        for j in range(NVEC):
                            sl = pl.ds(j * LANES, LANES)
                            obuf[b, i, sl] = ibuf[b, i, sl] * gvecs[j]

                    plsc.parallel_loop(lo, hi, unroll=2)(row)

            pltpu.async_copy(obuf.at[b], out_hbm.at[pl.ds(chunk_lo, C3)],
                             osems[b])

            @pl.when(kb + 2 < nch)
            def _next_in(b=b, chunk_lo=chunk_lo):
                pltpu.async_copy(flat_hbm.at[pl.ds(chunk_lo + 2 * C3, C3)],
                                 ibuf.at[b], isems[b])

    for b in range(2):
        pltpu.make_async_copy(
            obuf.at[b], out_hbm.at[pl.ds(base + (nch - 2 + b) * C3, C3)],
            osems[b]).wait()


def kernel(flat, cu_seqlens, W):
    cu_pad = jnp.concatenate([
        cu_seqlens.astype(jnp.int32),
        jnp.full((CU_PAD - NSEG - 1,), TT, dtype=jnp.int32),
    ])

    mesh1 = plsc.VectorSubcoreMesh(core_axis_name="c", subcore_axis_name="s")
    pass1 = functools.partial(
        pl.kernel,
        out_type=jax.ShapeDtypeStruct((NC, NSEG, DD), jnp.float32),
        mesh=mesh1,
        compiler_params=_sc_compiler_params(),
        scratch_types=[
            pltpu.VMEM((2, C1, DD), jnp.float32),
            pltpu.VMEM((NSEG, DD), jnp.float32),
            pltpu.VMEM((CU_PAD,), jnp.int32),
            pltpu.VMEM((C1 // 128, 128), jnp.int32),
            pltpu.VMEM_SHARED((NSEG, DD), jnp.float32),
            pltpu.SemaphoreType.DMA,
            pltpu.SemaphoreType.DMA,
        ],
    )(_pass1_body)
    partials = pass1(flat, cu_pad)

    mesh3 = plsc.VectorSubcoreMesh(core_axis_name="c", subcore_axis_name="s")
    pass3 = functools.partial(
        pl.kernel,
        out_type=jax.ShapeDtypeStruct((TT, DD), jnp.float32),
        mesh=mesh3,
        compiler_params=_sc_compiler_params(),
        scratch_types=[
            pltpu.VMEM((2, C3, DD), jnp.float32),
            pltpu.VMEM((2, C3, DD), jnp.float32),
            pltpu.VMEM((NSEG, DD), jnp.float32),
            pltpu.VMEM((CU_PAD,), jnp.int32),
            pltpu.VMEM((NC, NSEG, DD), jnp.float32),
            pltpu.VMEM((DD, DD), jnp.float32),
            pltpu.VMEM((DD,), jnp.float32),
            pltpu.VMEM((DD,), jnp.float32),
            pltpu.VMEM_SHARED((NSEG, DD), jnp.float32),
            pltpu.SemaphoreType.DMA,
            pltpu.SemaphoreType.DMA,
            pltpu.SemaphoreType.DMA,
            pltpu.SemaphoreType.DMA,
        ],
    )(_pass3_body)
    return pass3(flat, cu_pad, partials, W)


# speedup vs baseline: 1.2231x; 1.0882x over previous
import dataclasses
import functools

import jax
import jax.numpy as jnp
from jax import lax
from jax.experimental import pallas as pl
from jax.experimental.pallas import tpu as pltpu
from jax.experimental.pallas import tpu_sc as plsc

TT = 32768
DD = 128
NSEG = 16
LANES = 16
NVEC = DD // LANES
NC = 2
NS = 16
NW = NC * NS
RPW = TT // NW
C1 = 256
C3 = 128
CU_PAD = 32


def _sc_compiler_params():
    cp = pltpu.CompilerParams()
    if "needs_layout_passes" in pltpu.CompilerParams.__dataclass_fields__:
        cp = dataclasses.replace(cp, needs_layout_passes=False)
    return cp


def _cu_scalars(cu_v):
    cu_vec = cu_v[pl.ds(0, LANES)]
    lane_ids = lax.iota(jnp.int32, LANES)
    cus = [jnp.int32(0)]
    for s in range(1, NSEG):
        cus.append(jnp.sum(jnp.where(lane_ids == s, cu_vec, 0)))
    cus.append(jnp.int32(TT))
    return cus


def _pass1_body(flat_hbm, cu_hbm, part_hbm, buf, zbuf, cu_v, idx_v,
                acc_sh, isem0, isem1):
    isems = (isem0, isem1)
    cid = lax.axis_index("c")
    tid = lax.axis_index("s")
    base = (cid * NS + tid) * RPW
    nch = RPW // C1
    pltpu.async_copy(flat_hbm.at[pl.ds(base, C1)], buf.at[0], isems[0])
    pltpu.async_copy(flat_hbm.at[pl.ds(base + C1, C1)], buf.at[1], isems[1])
    pltpu.sync_copy(cu_hbm, cu_v)
    cus = _cu_scalars(cu_v)
    zero = jnp.zeros((LANES,), jnp.float32)
    for j in range(NVEC):
        for s in range(NSEG):
            zbuf[s, pl.ds(j * LANES, LANES)] = zero

    @pl.when(tid == 0)
    def _zero_acc():
        pltpu.sync_copy(zbuf, acc_sh)

    plsc.subcore_barrier()
    lane_ids = lax.iota(jnp.int32, LANES)
    ngrp = C1 // LANES

    for k in range(nch):
        b = k % 2
        chunk_lo = base + k * C1
        pltpu.make_async_copy(flat_hbm.at[pl.ds(chunk_lo, C1)],
                              buf.at[b], isems[b]).wait()

        @pl.loop(0, ngrp)
        def _grp(g, chunk_lo=chunk_lo):
            row_ids = lane_ids + (chunk_lo + g * LANES)
            seg = jnp.zeros((LANES,), jnp.int32)
            for s in range(1, NSEG):
                seg = seg + jnp.where(row_ids >= cus[s], 1, 0)
            idx_v[g // 8, pl.ds((g % 8) * LANES, LANES)] = seg

        for h in range(C1 // 128):
            pltpu.sync_copy(buf.at[b, pl.ds(h * 128, 128)],
                            acc_sh.at[idx_v.at[h]], add=True)
        if k + 2 < nch:
            pltpu.async_copy(flat_hbm.at[pl.ds(chunk_lo + 2 * C1, C1)],
                             buf.at[b], isems[b])
    plsc.subcore_barrier()

    @pl.when(tid == 0)
    def _emit():
        pltpu.sync_copy(acc_sh, part_hbm.at[cid])


def _cu_at_dyn(cu_v, idx):
    lane_ids = lax.iota(jnp.int32, LANES)
    v0 = cu_v[pl.ds(0, LANES)]
    v1 = cu_v[pl.ds(LANES, LANES)]
    return (jnp.sum(jnp.where(lane_ids == idx, v0, 0))
            + jnp.sum(jnp.where(lane_ids == idx - LANES, v1, 0)))


def _pass3_body(flat_hbm, cu_hbm, part_hbm, w_hbm, out_hbm,
                ibuf, obuf, gate_v, cu_v, pbuf, wbuf, pooled_v, grow_v,
                gate_sh, isem0, isem1, osem0, osem1):
    isems = (isem0, isem1)
    osems = (osem0, osem1)
    cid = lax.axis_index("c")
    tid = lax.axis_index("s")
    wid = cid * NS + tid
    base = wid * RPW
    nch = RPW // C3
    pltpu.async_copy(flat_hbm.at[pl.ds(base, C3)],
                     ibuf.at[pl.ds(0, C3)], isems[0])
    pltpu.async_copy(flat_hbm.at[pl.ds(base + C3, C3)],
                     ibuf.at[pl.ds(C3, C3)], isems[1])
    pltpu.sync_copy(cu_hbm, cu_v)
    cus = _cu_scalars(cu_v)

    pltpu.sync_copy(part_hbm, pbuf)
    pltpu.sync_copy(w_hbm, wbuf)
    zero = jnp.zeros((LANES,), jnp.float32)
    psum = tuple(pbuf[0, tid, pl.ds(j * LANES, LANES)]
                 + pbuf[1, tid, pl.ds(j * LANES, LANES)]
                 for j in range(NVEC))
    cnt = _cu_at_dyn(cu_v, tid + 1) - _cu_at_dyn(cu_v, tid)
    cnt_vec = jnp.zeros((LANES,), jnp.float32) + cnt.astype(jnp.float32)
    inv = 1.0 / jnp.maximum(cnt_vec, 1.0)
    for j in range(NVEC):
        pooled_v[pl.ds(j * LANES, LANES)] = psum[j] * inv

    def _fma(k, c):
        pk = plsc.load_gather(pooled_v, [jnp.full((LANES,), 0, jnp.int32) + k])
        return tuple(c[j] + pk * wbuf[k, pl.ds(j * LANES, LANES)]
                     for j in range(NVEC))

    macc = plsc.parallel_loop(0, DD, unroll=2,
                              carry=tuple(zero for _ in range(NVEC)))(_fma)
    for j in range(NVEC):
        g = 1.0 / (1.0 + jnp.exp(-macc[j]))
        grow_v[pl.ds(j * LANES, LANES)] = g
    pltpu.sync_copy(grow_v, gate_sh.at[tid])
    plsc.subcore_barrier()
    pltpu.sync_copy(gate_sh, gate_v)

    @pl.loop(0, nch)
    def _chunks(k):
        par = lax.rem(k, 2)
        roff = par * C3
        chunk_lo = base + k * C3
        for b in range(2):
            @pl.when(par == b)
            def _wait_in(b=b, chunk_lo=chunk_lo):
                pltpu.make_async_copy(flat_hbm.at[pl.ds(chunk_lo, C3)],
                                      ibuf.at[pl.ds(b * C3, C3)],
                                      isems[b]).wait()

            @pl.when((par == b) & (k >= 2))
            def _wait_out(b=b, chunk_lo=chunk_lo):
                pltpu.make_async_copy(
                    obuf.at[pl.ds(b * C3, C3)],
                    out_hbm.at[pl.ds(chunk_lo - 2 * C3, C3)],
                    osems[b]).wait()

        for s in range(NSEG):
            lo = jnp.clip(cus[s], chunk_lo, chunk_lo + C3) - chunk_lo
            hi = jnp.clip(cus[s + 1], chunk_lo, chunk_lo + C3) - chunk_lo

            @pl.when(hi > lo)
            def _run(s=s, lo=lo, hi=hi, roff=roff):
                gvecs = [gate_v[s, pl.ds(j * LANES, LANES)]
                         for j in range(NVEC)]

                def row(i):
                    for j in range(NVEC):
                        sl = pl.ds(j * LANES, LANES)
                        obuf[roff + i, sl] = ibuf[roff + i, sl] * gvecs[j]

                plsc.parallel_loop(lo, hi, unroll=4)(row)

        for b in range(2):
            @pl.when(par == b)
            def _issue(b=b, chunk_lo=chunk_lo):
                pltpu.async_copy(obuf.at[pl.ds(b * C3, C3)],
                                 out_hbm.at[pl.ds(chunk_lo, C3)], osems[b])

            @pl.when((par == b) & (k + 2 < nch))
            def _next_in(b=b, chunk_lo=chunk_lo):
                pltpu.async_copy(flat_hbm.at[pl.ds(chunk_lo + 2 * C3, C3)],
                                 ibuf.at[pl.ds(b * C3, C3)], isems[b])

    for b in range(2):
        pltpu.make_async_copy(
            obuf.at[pl.ds(b * C3, C3)],
            out_hbm.at[pl.ds(base + (nch - 2 + b) * C3, C3)],
            osems[b]).wait()


def kernel(flat, cu_seqlens, W):
    cu_pad = jnp.concatenate([
        cu_seqlens.astype(jnp.int32),
        jnp.full((CU_PAD - NSEG - 1,), TT, dtype=jnp.int32),
    ])

    mesh1 = plsc.VectorSubcoreMesh(core_axis_name="c", subcore_axis_name="s")
    pass1 = functools.partial(
        pl.kernel,
        out_type=jax.ShapeDtypeStruct((NC, NSEG, DD), jnp.float32),
        mesh=mesh1,
        compiler_params=_sc_compiler_params(),
        scratch_types=[
            pltpu.VMEM((2, C1, DD), jnp.float32),
            pltpu.VMEM((NSEG, DD), jnp.float32),
            pltpu.VMEM((CU_PAD,), jnp.int32),
            pltpu.VMEM((C1 // 128, 128), jnp.int32),
            pltpu.VMEM_SHARED((NSEG, DD), jnp.float32),
            pltpu.SemaphoreType.DMA,
            pltpu.SemaphoreType.DMA,
        ],
    )(_pass1_body)
    partials = pass1(flat, cu_pad)

    mesh3 = plsc.VectorSubcoreMesh(core_axis_name="c", subcore_axis_name="s")
    pass3 = functools.partial(
        pl.kernel,
        out_type=jax.ShapeDtypeStruct((TT, DD), jnp.float32),
        mesh=mesh3,
        compiler_params=_sc_compiler_params(),
        scratch_types=[
            pltpu.VMEM((2 * C3, DD), jnp.float32),
            pltpu.VMEM((2 * C3, DD), jnp.float32),
            pltpu.VMEM((NSEG, DD), jnp.float32),
            pltpu.VMEM((CU_PAD,), jnp.int32),
            pltpu.VMEM((NC, NSEG, DD), jnp.float32),
            pltpu.VMEM((DD, DD), jnp.float32),
            pltpu.VMEM((DD,), jnp.float32),
            pltpu.VMEM((DD,), jnp.float32),
            pltpu.VMEM_SHARED((NSEG, DD), jnp.float32),
            pltpu.SemaphoreType.DMA,
            pltpu.SemaphoreType.DMA,
            pltpu.SemaphoreType.DMA,
            pltpu.SemaphoreType.DMA,
        ],
    )(_pass3_body)
    return pass3(flat, cu_pad, partials, W)


# speedup vs baseline: 1.5488x; 1.2662x over previous
import dataclasses
import functools

import jax
import jax.numpy as jnp
from jax import lax
from jax.experimental import pallas as pl
from jax.experimental.pallas import tpu as pltpu
from jax.experimental.pallas import tpu_sc as plsc

TT = 32768
DD = 128
NSEG = 16
LANES = 16
NVEC = DD // LANES
NC = 2
NS = 16
NW = NC * NS
RPW = TT // NW
C1 = 256
C3 = 128
CU_PAD = 32


def _sc_compiler_params():
    cp = pltpu.CompilerParams()
    if "needs_layout_passes" in pltpu.CompilerParams.__dataclass_fields__:
        cp = dataclasses.replace(cp, needs_layout_passes=False)
    return cp


def _cu_scalars(cu_v):
    cu_vec = cu_v[pl.ds(0, LANES)]
    lane_ids = lax.iota(jnp.int32, LANES)
    cus = [jnp.int32(0)]
    for s in range(1, NSEG):
        cus.append(jnp.sum(jnp.where(lane_ids == s, cu_vec, 0)))
    cus.append(jnp.int32(TT))
    return cus


def _pass1_body(flat_hbm, cu_hbm, part_hbm, buf, zbuf, cu_v, idx_v,
                acc_sh, isem0, isem1):
    isems = (isem0, isem1)
    cid = lax.axis_index("c")
    tid = lax.axis_index("s")
    base = (cid * NS + tid) * RPW
    nch = RPW // C1
    pltpu.async_copy(flat_hbm.at[pl.ds(base, C1)], buf.at[0], isems[0])
    pltpu.async_copy(flat_hbm.at[pl.ds(base + C1, C1)], buf.at[1], isems[1])
    pltpu.sync_copy(cu_hbm, cu_v)
    cus = _cu_scalars(cu_v)
    zero = jnp.zeros((LANES,), jnp.float32)
    for j in range(NVEC):
        for s in range(NSEG):
            zbuf[s, pl.ds(j * LANES, LANES)] = zero

    @pl.when(tid == 0)
    def _zero_acc():
        pltpu.sync_copy(zbuf, acc_sh)

    plsc.subcore_barrier()
    lane_ids = lax.iota(jnp.int32, LANES)
    ngrp = C1 // LANES

    for k in range(nch):
        b = k % 2
        chunk_lo = base + k * C1
        pltpu.make_async_copy(flat_hbm.at[pl.ds(chunk_lo, C1)],
                              buf.at[b], isems[b]).wait()

        @pl.loop(0, ngrp)
        def _grp(g, chunk_lo=chunk_lo):
            row_ids = lane_ids + (chunk_lo + g * LANES)
            seg = jnp.zeros((LANES,), jnp.int32)
            for s in range(1, NSEG):
                seg = seg + jnp.where(row_ids >= cus[s], 1, 0)
            idx_v[g // 8, pl.ds((g % 8) * LANES, LANES)] = seg

        for h in range(C1 // 128):
            pltpu.sync_copy(buf.at[b, pl.ds(h * 128, 128)],
                            acc_sh.at[idx_v.at[h]], add=True)
        if k + 2 < nch:
            pltpu.async_copy(flat_hbm.at[pl.ds(chunk_lo + 2 * C1, C1)],
                             buf.at[b], isems[b])
    plsc.subcore_barrier()

    @pl.when(tid == 0)
    def _emit():
        pltpu.sync_copy(acc_sh, part_hbm.at[cid])


def _cu_at_dyn(cu_v, idx):
    lane_ids = lax.iota(jnp.int32, LANES)
    v0 = cu_v[pl.ds(0, LANES)]
    v1 = cu_v[pl.ds(LANES, LANES)]
    return (jnp.sum(jnp.where(lane_ids == idx, v0, 0))
            + jnp.sum(jnp.where(lane_ids == idx - LANES, v1, 0)))


def _pass3_body(flat_hbm, cu_hbm, part_hbm, w_hbm, out_hbm,
                ibuf, obuf, gate_v, cu_v, pbuf, wbuf, pooled_v, grow_v,
                gate_sh, isem0, isem1, osem0, osem1):
    isems = (isem0, isem1)
    osems = (osem0, osem1)
    cid = lax.axis_index("c")
    tid = lax.axis_index("s")
    wid = cid * NS + tid
    base = wid * RPW
    nch = RPW // C3
    pltpu.async_copy(flat_hbm.at[pl.ds(base, C3)],
                     ibuf.at[pl.ds(0, C3)], isems[0])
    pltpu.async_copy(flat_hbm.at[pl.ds(base + C3, C3)],
                     ibuf.at[pl.ds(C3, C3)], isems[1])
    pw = pltpu.async_copy(w_hbm, wbuf, osems[1])
    pp = pltpu.async_copy(part_hbm, pbuf, osems[0])
    pltpu.sync_copy(cu_hbm, cu_v)
    pp.wait()
    pw.wait()
    zero = jnp.zeros((LANES,), jnp.float32)
    psum = tuple(pbuf[0, tid, pl.ds(j * LANES, LANES)]
                 + pbuf[1, tid, pl.ds(j * LANES, LANES)]
                 for j in range(NVEC))
    cnt = _cu_at_dyn(cu_v, tid + 1) - _cu_at_dyn(cu_v, tid)
    cnt_vec = jnp.zeros((LANES,), jnp.float32) + cnt.astype(jnp.float32)
    inv = 1.0 / jnp.maximum(cnt_vec, 1.0)
    for j in range(NVEC):
        pooled_v[pl.ds(j * LANES, LANES)] = psum[j] * inv

    def _fma(k, c):
        pk = plsc.load_gather(pooled_v, [jnp.full((LANES,), 0, jnp.int32) + k])
        return tuple(c[j] + pk * wbuf[k, pl.ds(j * LANES, LANES)]
                     for j in range(NVEC))

    macc = plsc.parallel_loop(0, DD, unroll=2,
                              carry=tuple(zero for _ in range(NVEC)))(_fma)
    for j in range(NVEC):
        g = 1.0 / (1.0 + jnp.exp(-macc[j]))
        grow_v[pl.ds(j * LANES, LANES)] = g
    pltpu.sync_copy(grow_v, gate_sh.at[tid])
    plsc.subcore_barrier()
    pltpu.sync_copy(gate_sh, gate_v)

    @pl.loop(0, nch)
    def _chunks(k):
        par = lax.rem(k, 2)
        roff = par * C3
        chunk_lo = base + k * C3
        for b in range(2):
            @pl.when(par == b)
            def _wait_in(b=b, chunk_lo=chunk_lo):
                pltpu.make_async_copy(flat_hbm.at[pl.ds(chunk_lo, C3)],
                                      ibuf.at[pl.ds(b * C3, C3)],
                                      isems[b]).wait()

            @pl.when((par == b) & (k >= 2))
            def _wait_out(b=b, chunk_lo=chunk_lo):
                pltpu.make_async_copy(
                    obuf.at[pl.ds(b * C3, C3)],
                    out_hbm.at[pl.ds(chunk_lo - 2 * C3, C3)],
                    osems[b]).wait()

        @pl.loop(0, NSEG)
        def _seg(s, chunk_lo=chunk_lo, roff=roff):
            c_lo = _cu_at_dyn(cu_v, s)
            c_hi = _cu_at_dyn(cu_v, s + 1)
            lo = jnp.clip(c_lo, chunk_lo, chunk_lo + C3) - chunk_lo
            hi = jnp.clip(c_hi, chunk_lo, chunk_lo + C3) - chunk_lo

            @pl.when(hi > lo)
            def _run():
                gvecs = [gate_v[s, pl.ds(j * LANES, LANES)]
                         for j in range(NVEC)]

                def row(i):
                    for j in range(NVEC):
                        sl = pl.ds(j * LANES, LANES)
                        obuf[roff + i, sl] = ibuf[roff + i, sl] * gvecs[j]

                plsc.parallel_loop(lo, hi, unroll=4)(row)

        for b in range(2):
            @pl.when(par == b)
            def _issue(b=b, chunk_lo=chunk_lo):
                pltpu.async_copy(obuf.at[pl.ds(b * C3, C3)],
                                 out_hbm.at[pl.ds(chunk_lo, C3)], osems[b])

            @pl.when((par == b) & (k + 2 < nch))
            def _next_in(b=b, chunk_lo=chunk_lo):
                pltpu.async_copy(flat_hbm.at[pl.ds(chunk_lo + 2 * C3, C3)],
                                 ibuf.at[pl.ds(b * C3, C3)], isems[b])

    for b in range(2):
        pltpu.make_async_copy(
            obuf.at[pl.ds(b * C3, C3)],
            out_hbm.at[pl.ds(base + (nch - 2 + b) * C3, C3)],
            osems[b]).wait()


def kernel(flat, cu_seqlens, W):
    cu_pad = jnp.concatenate([
        cu_seqlens.astype(jnp.int32),
        jnp.full((CU_PAD - NSEG - 1,), TT, dtype=jnp.int32),
    ])

    mesh1 = plsc.VectorSubcoreMesh(core_axis_name="c", subcore_axis_name="s")
    pass1 = functools.partial(
        pl.kernel,
        out_type=jax.ShapeDtypeStruct((NC, NSEG, DD), jnp.float32),
        mesh=mesh1,
        compiler_params=_sc_compiler_params(),
        scratch_types=[
            pltpu.VMEM((2, C1, DD), jnp.float32),
            pltpu.VMEM((NSEG, DD), jnp.float32),
            pltpu.VMEM((CU_PAD,), jnp.int32),
            pltpu.VMEM((C1 // 128, 128), jnp.int32),
            pltpu.VMEM_SHARED((NSEG, DD), jnp.float32),
            pltpu.SemaphoreType.DMA,
            pltpu.SemaphoreType.DMA,
        ],
    )(_pass1_body)
    partials = pass1(flat, cu_pad)

    mesh3 = plsc.VectorSubcoreMesh(core_axis_name="c", subcore_axis_name="s")
    pass3 = functools.partial(
        pl.kernel,
        out_type=jax.ShapeDtypeStruct((TT, DD), jnp.float32),
        mesh=mesh3,
        compiler_params=_sc_compiler_params(),
        scratch_types=[
            pltpu.VMEM((2 * C3, DD), jnp.float32),
            pltpu.VMEM((2 * C3, DD), jnp.float32),
            pltpu.VMEM((NSEG, DD), jnp.float32),
            pltpu.VMEM((CU_PAD,), jnp.int32),
            pltpu.VMEM((NC, NSEG, DD), jnp.float32),
            pltpu.VMEM((DD, DD), jnp.float32),
            pltpu.VMEM((DD,), jnp.float32),
            pltpu.VMEM((DD,), jnp.float32),
            pltpu.VMEM_SHARED((NSEG, DD), jnp.float32),
            pltpu.SemaphoreType.DMA,
            pltpu.SemaphoreType.DMA,
            pltpu.SemaphoreType.DMA,
            pltpu.SemaphoreType.DMA,
        ],
    )(_pass3_body)
    return pass3(flat, cu_pad, partials, W)


# speedup vs baseline: 1.7092x; 1.1036x over previous
import dataclasses
import functools

import jax
import jax.numpy as jnp
from jax import lax
from jax.experimental import pallas as pl
from jax.experimental.pallas import tpu as pltpu
from jax.experimental.pallas import tpu_sc as plsc

TT = 32768
DD = 128
NSEG = 16
LANES = 16
NVEC = DD // LANES
NC = 2
NS = 16
NW = NC * NS
RPW = TT // NW
C1 = 256
C3 = 128
CU_PAD = 32
SPLIT = 16384
RPW1 = SPLIT // NW
TCR = TT - SPLIT


def _sc_compiler_params():
    cp = pltpu.CompilerParams()
    if "needs_layout_passes" in pltpu.CompilerParams.__dataclass_fields__:
        cp = dataclasses.replace(cp, needs_layout_passes=False)
    return cp


def _cu_scalars(cu_v):
    cu_vec = cu_v[pl.ds(0, LANES)]
    lane_ids = lax.iota(jnp.int32, LANES)
    cus = [jnp.int32(0)]
    for s in range(1, NSEG):
        cus.append(jnp.sum(jnp.where(lane_ids == s, cu_vec, 0)))
    cus.append(jnp.int32(TT))
    return cus


def _pass1_body(flat_hbm, cu_hbm, part_hbm, buf, zbuf, cu_v, idx_v,
                acc_sh, isem0, isem1):
    isems = (isem0, isem1)
    cid = lax.axis_index("c")
    tid = lax.axis_index("s")
    base = (cid * NS + tid) * RPW1
    nch = RPW1 // C1
    pltpu.async_copy(flat_hbm.at[pl.ds(base, C1)], buf.at[0], isems[0])
    pltpu.async_copy(flat_hbm.at[pl.ds(base + C1, C1)], buf.at[1], isems[1])
    pltpu.sync_copy(cu_hbm, cu_v)
    cus = _cu_scalars(cu_v)
    zero = jnp.zeros((LANES,), jnp.float32)
    for j in range(NVEC):
        for s in range(NSEG):
            zbuf[s, pl.ds(j * LANES, LANES)] = zero

    @pl.when(tid == 0)
    def _zero_acc():
        pltpu.sync_copy(zbuf, acc_sh)

    plsc.subcore_barrier()
    lane_ids = lax.iota(jnp.int32, LANES)
    ngrp = C1 // LANES

    for k in range(nch):
        b = k % 2
        chunk_lo = base + k * C1
        pltpu.make_async_copy(flat_hbm.at[pl.ds(chunk_lo, C1)],
                              buf.at[b], isems[b]).wait()

        @pl.loop(0, ngrp)
        def _grp(g, chunk_lo=chunk_lo):
            row_ids = lane_ids + (chunk_lo + g * LANES)
            seg = jnp.zeros((LANES,), jnp.int32)
            for s in range(1, NSEG):
                seg = seg + jnp.where(row_ids >= cus[s], 1, 0)
            idx_v[g // 8, pl.ds((g % 8) * LANES, LANES)] = seg

        for h in range(C1 // 128):
            pltpu.sync_copy(buf.at[b, pl.ds(h * 128, 128)],
                            acc_sh.at[idx_v.at[h]], add=True)
        if k + 2 < nch:
            pltpu.async_copy(flat_hbm.at[pl.ds(chunk_lo + 2 * C1, C1)],
                             buf.at[b], isems[b])
    plsc.subcore_barrier()

    @pl.when(tid == 0)
    def _emit():
        pltpu.sync_copy(acc_sh, part_hbm.at[cid])


def _pass1_tc_body(cu_smem, flat_ref, part_ref):
    rows1 = lax.broadcasted_iota(jnp.int32, (NSEG, 1), 0)
    lower = jnp.zeros((NSEG, 1), jnp.int32)
    upper = jnp.zeros((NSEG, 1), jnp.int32)
    for s in range(NSEG):
        lower = jnp.where(rows1 == s, cu_smem[s], lower)
        upper = jnp.where(rows1 == s, cu_smem[s + 1], upper)
    cols = lax.broadcasted_iota(jnp.int32, (NSEG, TCR), 1) + SPLIT
    sel = ((cols >= lower) & (cols < upper)).astype(jnp.float32)
    part_ref[...] = jnp.dot(sel, flat_ref[...],
                            preferred_element_type=jnp.float32)


def _cu_at_dyn(cu_v, idx):
    lane_ids = lax.iota(jnp.int32, LANES)
    v0 = cu_v[pl.ds(0, LANES)]
    v1 = cu_v[pl.ds(LANES, LANES)]
    return (jnp.sum(jnp.where(lane_ids == idx, v0, 0))
            + jnp.sum(jnp.where(lane_ids == idx - LANES, v1, 0)))


def _pass3_body(flat_hbm, cu_hbm, part_hbm, ptc_hbm, w_hbm, out_hbm,
                ibuf, obuf, gate_v, cu_v, pbuf, ptc, wbuf, pooled_v, grow_v,
                gate_sh, isem0, isem1, osem0, osem1, psem):
    isems = (isem0, isem1)
    osems = (osem0, osem1)
    cid = lax.axis_index("c")
    tid = lax.axis_index("s")
    wid = cid * NS + tid
    base = wid * RPW
    nch = RPW // C3
    pltpu.async_copy(flat_hbm.at[pl.ds(base, C3)],
                     ibuf.at[pl.ds(0, C3)], isems[0])
    pltpu.async_copy(flat_hbm.at[pl.ds(base + C3, C3)],
                     ibuf.at[pl.ds(C3, C3)], isems[1])
    pw = pltpu.async_copy(w_hbm, wbuf, osems[1])
    pp = pltpu.async_copy(part_hbm, pbuf, osems[0])
    pt = pltpu.async_copy(ptc_hbm, ptc, psem)
    pltpu.sync_copy(cu_hbm, cu_v)
    pp.wait()
    pw.wait()
    pt.wait()
    zero = jnp.zeros((LANES,), jnp.float32)
    psum = tuple(pbuf[0, tid, pl.ds(j * LANES, LANES)]
                 + pbuf[1, tid, pl.ds(j * LANES, LANES)]
                 + ptc[tid, pl.ds(j * LANES, LANES)]
                 for j in range(NVEC))
    cnt = _cu_at_dyn(cu_v, tid + 1) - _cu_at_dyn(cu_v, tid)
    cnt_vec = jnp.zeros((LANES,), jnp.float32) + cnt.astype(jnp.float32)
    inv = 1.0 / jnp.maximum(cnt_vec, 1.0)
    for j in range(NVEC):
        pooled_v[pl.ds(j * LANES, LANES)] = psum[j] * inv

    def _fma(k, c):
        pk = plsc.load_gather(pooled_v, [jnp.full((LANES,), 0, jnp.int32) + k])
        return tuple(c[j] + pk * wbuf[k, pl.ds(j * LANES, LANES)]
                     for j in range(NVEC))

    macc = plsc.parallel_loop(0, DD, unroll=2,
                              carry=tuple(zero for _ in range(NVEC)))(_fma)
    for j in range(NVEC):
        g = 1.0 / (1.0 + jnp.exp(-macc[j]))
        grow_v[pl.ds(j * LANES, LANES)] = g
    pltpu.sync_copy(grow_v, gate_sh.at[tid])
    plsc.subcore_barrier()
    pltpu.sync_copy(gate_sh, gate_v)

    @pl.loop(0, nch)
    def _chunks(k):
        par = lax.rem(k, 2)
        roff = par * C3
        chunk_lo = base + k * C3
        for b in range(2):
            @pl.when(par == b)
            def _wait_in(b=b, chunk_lo=chunk_lo):
                pltpu.make_async_copy(flat_hbm.at[pl.ds(chunk_lo, C3)],
                                      ibuf.at[pl.ds(b * C3, C3)],
                                      isems[b]).wait()

            @pl.when((par == b) & (k >= 2))
            def _wait_out(b=b, chunk_lo=chunk_lo):
                pltpu.make_async_copy(
                    obuf.at[pl.ds(b * C3, C3)],
                    out_hbm.at[pl.ds(chunk_lo - 2 * C3, C3)],
                    osems[b]).wait()

        @pl.loop(0, NSEG)
        def _seg(s, chunk_lo=chunk_lo, roff=roff):
            c_lo = _cu_at_dyn(cu_v, s)
            c_hi = _cu_at_dyn(cu_v, s + 1)
            lo = jnp.clip(c_lo, chunk_lo, chunk_lo + C3) - chunk_lo
            hi = jnp.clip(c_hi, chunk_lo, chunk_lo + C3) - chunk_lo

            @pl.when(hi > lo)
            def _run():
                gvecs = [gate_v[s, pl.ds(j * LANES, LANES)]
                         for j in range(NVEC)]

                def row(i):
                    for j in range(NVEC):
                        sl = pl.ds(j * LANES, LANES)
                        obuf[roff + i, sl] = ibuf[roff + i, sl] * gvecs[j]

                plsc.parallel_loop(lo, hi, unroll=4)(row)

        for b in range(2):
            @pl.when(par == b)
            def _issue(b=b, chunk_lo=chunk_lo):
                pltpu.async_copy(obuf.at[pl.ds(b * C3, C3)],
                                 out_hbm.at[pl.ds(chunk_lo, C3)], osems[b])

            @pl.when((par == b) & (k + 2 < nch))
            def _next_in(b=b, chunk_lo=chunk_lo):
                pltpu.async_copy(flat_hbm.at[pl.ds(chunk_lo + 2 * C3, C3)],
                                 ibuf.at[pl.ds(b * C3, C3)], isems[b])

    for b in range(2):
        pltpu.make_async_copy(
            obuf.at[pl.ds(b * C3, C3)],
            out_hbm.at[pl.ds(base + (nch - 2 + b) * C3, C3)],
            osems[b]).wait()


def kernel(flat, cu_seqlens, W):
    cu_pad = jnp.concatenate([
        cu_seqlens.astype(jnp.int32),
        jnp.full((CU_PAD - NSEG - 1,), TT, dtype=jnp.int32),
    ])

    mesh1 = plsc.VectorSubcoreMesh(core_axis_name="c", subcore_axis_name="s")
    pass1 = functools.partial(
        pl.kernel,
        out_type=jax.ShapeDtypeStruct((NC, NSEG, DD), jnp.float32),
        mesh=mesh1,
        compiler_params=_sc_compiler_params(),
        scratch_types=[
            pltpu.VMEM((2, C1, DD), jnp.float32),
            pltpu.VMEM((NSEG, DD), jnp.float32),
            pltpu.VMEM((CU_PAD,), jnp.int32),
            pltpu.VMEM((C1 // 128, 128), jnp.int32),
            pltpu.VMEM_SHARED((NSEG, DD), jnp.float32),
            pltpu.SemaphoreType.DMA,
            pltpu.SemaphoreType.DMA,
        ],
    )(_pass1_body)
    partials = pass1(flat, cu_pad)

    part_tc = pl.pallas_call(
        _pass1_tc_body,
        grid=(1,),
        out_shape=jax.ShapeDtypeStruct((NSEG, DD), jnp.float32),
        in_specs=[
            pl.BlockSpec(memory_space=pltpu.SMEM),
            pl.BlockSpec((TCR, DD), lambda i: (SPLIT // TCR, 0)),
        ],
        out_specs=pl.BlockSpec((NSEG, DD), lambda i: (0, 0)),
    )(cu_pad, flat)

    mesh3 = plsc.VectorSubcoreMesh(core_axis_name="c", subcore_axis_name="s")
    pass3 = functools.partial(
        pl.kernel,
        out_type=jax.ShapeDtypeStruct((TT, DD), jnp.float32),
        mesh=mesh3,
        compiler_params=_sc_compiler_params(),
        scratch_types=[
            pltpu.VMEM((2 * C3, DD), jnp.float32),
            pltpu.VMEM((2 * C3, DD), jnp.float32),
            pltpu.VMEM((NSEG, DD), jnp.float32),
            pltpu.VMEM((CU_PAD,), jnp.int32),
            pltpu.VMEM((NC, NSEG, DD), jnp.float32),
            pltpu.VMEM((NSEG, DD), jnp.float32),
            pltpu.VMEM((DD, DD), jnp.float32),
            pltpu.VMEM((DD,), jnp.float32),
            pltpu.VMEM((DD,), jnp.float32),
            pltpu.VMEM_SHARED((NSEG, DD), jnp.float32),
            pltpu.SemaphoreType.DMA,
            pltpu.SemaphoreType.DMA,
            pltpu.SemaphoreType.DMA,
            pltpu.SemaphoreType.DMA,
            pltpu.SemaphoreType.DMA,
        ],
    )(_pass3_body)
    return pass3(flat, cu_pad, partials, part_tc, W)
